# baseline (speedup 1.0000x reference)
import functools

import jax
import jax.numpy as jnp
from jax.experimental import pallas as pl
from jax.experimental.pallas import tpu as pltpu

_FILL = -1000000000.0


def _l2_band_kernel(lhs_ref, rhs_ref, out_ref, body_ref, tail_ref, sem,
                    *, bb, tail):
    B, N, N_ = out_ref.shape
    body_cols = N_ - tail

    body_ref[...] = jnp.full(body_ref.shape, _FILL, dtype=body_ref.dtype)

    diff = lhs_ref[...] - rhs_ref[...]
    sim = jnp.sqrt(jnp.sum(diff * diff, axis=-1))
    row = jax.lax.broadcasted_iota(jnp.int32, (B, N, tail), 1)
    col = jax.lax.broadcasted_iota(jnp.int32, (B, N, tail), 2)
    mask = col == row + (tail - N)
    tail_ref[...] = jnp.where(mask, sim[:, :, None], jnp.float32(_FILL))

    copies = [
        pltpu.make_async_copy(
            body_ref,
            out_ref.at[pl.ds(j * bb, bb), :, pl.ds(0, body_cols)],
            sem,
        )
        for j in range(B // bb)
    ]
    copies.append(
        pltpu.make_async_copy(
            tail_ref, out_ref.at[:, :, pl.ds(body_cols, tail)], sem
        )
    )
    for c in copies:
        c.start()
    for c in copies:
        c.wait()


def kernel(lhs, rhs):
    B, N, dim = lhs.shape
    N_ = rhs.shape[1]
    bb = 4
    tail = 128
    tail_block_idx = N_ // N - 1

    body = functools.partial(_l2_band_kernel, bb=bb, tail=tail)
    return pl.pallas_call(
        body,
        grid=(1,),
        in_specs=[
            pl.BlockSpec((B, N, dim), lambda i: (0, 0, 0)),
            pl.BlockSpec((B, N, dim), lambda i: (0, tail_block_idx, 0)),
        ],
        out_specs=pl.BlockSpec(memory_space=pltpu.MemorySpace.HBM),
        out_shape=jax.ShapeDtypeStruct((B, N, N_), lhs.dtype),
        scratch_shapes=[
            pltpu.MemorySpace.VMEM((bb, N, N_ - tail), jnp.float32),
            pltpu.MemorySpace.VMEM((B, N, tail), jnp.float32),
            pltpu.SemaphoreType.DMA,
        ],
    )(lhs, rhs)


# speedup vs baseline: 1.0155x; 1.0155x over previous
import functools

import jax
import jax.numpy as jnp
from jax.experimental import pallas as pl
from jax.experimental.pallas import tpu as pltpu

_FILL = -1000000000.0


def _l2_band_kernel(lhs_ref, rhs_ref, out_ref, body_ref, tail_ref, sem,
                    *, bb, tail):
    B, N, N_ = out_ref.shape
    body_cols = N_ - tail

    body_ref[...] = jnp.full(body_ref.shape, _FILL, dtype=body_ref.dtype)

    copies = [
        pltpu.make_async_copy(
            body_ref,
            out_ref.at[pl.ds(j * bb, bb), :, pl.ds(0, body_cols)],
            sem,
        )
        for j in range(B // bb)
    ]
    for c in copies:
        c.start()

    diff = lhs_ref[...] - rhs_ref[...]
    sim = jnp.sqrt(jnp.sum(diff * diff, axis=-1))
    row = jax.lax.broadcasted_iota(jnp.int32, (B, N, tail), 1)
    col = jax.lax.broadcasted_iota(jnp.int32, (B, N, tail), 2)
    mask = col == row + (tail - N)
    tail_ref[...] = jnp.where(mask, sim[:, :, None], jnp.float32(_FILL))

    tail_copy = pltpu.make_async_copy(
        tail_ref, out_ref.at[:, :, pl.ds(body_cols, tail)], sem
    )
    tail_copy.start()

    for c in copies:
        c.wait()
    tail_copy.wait()


def kernel(lhs, rhs):
    B, N, dim = lhs.shape
    N_ = rhs.shape[1]
    bb = 2
    tail = 128
    tail_block_idx = N_ // N - 1

    body = functools.partial(_l2_band_kernel, bb=bb, tail=tail)
    return pl.pallas_call(
        body,
        grid=(1,),
        in_specs=[
            pl.BlockSpec((B, N, dim), lambda i: (0, 0, 0)),
            pl.BlockSpec((B, N, dim), lambda i: (0, tail_block_idx, 0)),
        ],
        out_specs=pl.BlockSpec(memory_space=pltpu.MemorySpace.HBM),
        out_shape=jax.ShapeDtypeStruct((B, N, N_), lhs.dtype),
        scratch_shapes=[
            pltpu.MemorySpace.VMEM((bb, N, N_ - tail), jnp.float32),
            pltpu.MemorySpace.VMEM((B, N, tail), jnp.float32),
            pltpu.SemaphoreType.DMA,
        ],
    )(lhs, rhs)
